# baseline (speedup 1.0000x reference)
import functools

import jax
import jax.numpy as jnp
import numpy as np
from jax.experimental import pallas as pl
from jax.experimental.pallas import tpu as pltpu

_N_RBF = 16
_CUTOFF = 5.0
_R0 = 0.5


def _silu(x):
    return x * jax.nn.sigmoid(x)


def _body(pos_ref, posT_ref, w1t_ref, b1_ref, w2t_ref, b2_ref, w3t_ref,
          b3_ref, out_ref, *, br, n):
    i = pl.program_id(0)

    pos = pos_ref[...]
    posT = posT_ref[...]

    d2 = jnp.zeros((br, n), jnp.float32)
    for c in range(3):
        diff = pos[:, c:c + 1] - posT[c:c + 1, :]
        d2 = d2 + diff * diff
    dist = jnp.sqrt(d2 + 1e-12)
    mask = (dist > 1e-6) & (dist < _CUTOFF)
    w = jnp.where(mask, 0.5 * (1.0 + jnp.cos((np.pi / _CUTOFF) * dist)), 0.0)

    eta = 0.5 * _CUTOFF / _N_RBF
    inv2eta2 = 1.0 / (2.0 * eta * eta)
    centers = np.linspace(_R0, _CUTOFF, _N_RBF)

    cols = []
    for k in range(_N_RBF):
        delta = dist - np.float32(centers[k])
        ek = jnp.exp(delta * delta * np.float32(-inv2eta2))
        cols.append(jnp.sum(w * ek, axis=1, keepdims=True))
    feats = jnp.concatenate(cols, axis=1)

    h = _silu(jnp.dot(feats, w1t_ref[...],
                      preferred_element_type=jnp.float32) + b1_ref[...])
    h = _silu(jnp.dot(h, w2t_ref[...],
                      preferred_element_type=jnp.float32) + b2_ref[...])
    o = jnp.dot(h, w3t_ref[...], preferred_element_type=jnp.float32)
    part = jnp.sum(o) + np.float32(br) * b3_ref[0, 0]

    @pl.when(i == 0)
    def _():
        out_ref[...] = jnp.zeros((1, 1), jnp.float32)

    out_ref[...] += jnp.full((1, 1), part, jnp.float32)


@jax.jit
def _run(positions, W1, b1, W2, b2, W3, b3):
    n = positions.shape[0]
    br = 256
    grid = n // br
    posT = positions.T
    w1t = W1.T
    w2t = W2.T
    w3t = W3.T
    b1r = b1.reshape(1, -1)
    b2r = b2.reshape(1, -1)
    b3r = b3.reshape(1, 1)

    out = pl.pallas_call(
        functools.partial(_body, br=br, n=n),
        grid=(grid,),
        in_specs=[
            pl.BlockSpec((br, 3), lambda i: (i, 0)),
            pl.BlockSpec((3, n), lambda i: (0, 0)),
            pl.BlockSpec(w1t.shape, lambda i: (0, 0)),
            pl.BlockSpec(b1r.shape, lambda i: (0, 0)),
            pl.BlockSpec(w2t.shape, lambda i: (0, 0)),
            pl.BlockSpec(b2r.shape, lambda i: (0, 0)),
            pl.BlockSpec(w3t.shape, lambda i: (0, 0)),
            pl.BlockSpec(b3r.shape, lambda i: (0, 0)),
        ],
        out_specs=pl.BlockSpec((1, 1), lambda i: (0, 0)),
        out_shape=jax.ShapeDtypeStruct((1, 1), jnp.float32),
        compiler_params=pltpu.CompilerParams(
            dimension_semantics=("arbitrary",),
        ),
    )(positions, posT, w1t, b1r, w2t, b2r, w3t, b3r)
    return out[0, 0]


def kernel(positions, W1, b1, W2, b2, W3, b3):
    return _run(positions, W1, b1, W2, b2, W3, b3)


# speedup vs baseline: 2.2233x; 2.2233x over previous
import functools

import jax
import jax.numpy as jnp
import numpy as np
from jax.experimental import pallas as pl
from jax.experimental.pallas import tpu as pltpu

_N_RBF = 16
_CUTOFF = 5.0
_R0 = 0.5

_COS_COEF = [
    0.9999999998846174, -0.1973920874308756, 0.0064939389052478935,
    -8.545665775343764e-05, 6.024212492804692e-07, -2.6404630605029533e-09,
    7.800022166687994e-12, -1.452987492084503e-14,
]


def _silu(x):
    return x * jax.nn.sigmoid(x)


def _body(pos_ref, posT_ref, w1t_ref, b1_ref, w2t_ref, b2_ref, w3t_ref,
          b3_ref, out_ref, feat_ref, *, br, n):
    bi = pl.program_id(0)
    bj = pl.program_id(1)
    nb = n // br

    @pl.when(jnp.logical_and(bi == 0, bj == 0))
    def _():
        feat_ref[...] = jnp.zeros((n, _N_RBF), jnp.float32)

    @pl.when(bj >= bi)
    def _():
        pos = pos_ref[...]
        posT = posT_ref[...]

        d2 = jnp.full((br, br), 1e-12, jnp.float32)
        for c in range(3):
            diff = pos[:, c:c + 1] - posT[c:c + 1, :]
            d2 = d2 + diff * diff

        r = jax.lax.rsqrt(d2)
        r = r * (1.5 - (0.5 * d2) * (r * r))
        dist = d2 * r

        mask = (dist > 1e-6) & (dist < _CUTOFF)

        cpoly = jnp.full((br, br), np.float32(_COS_COEF[-1]), jnp.float32)
        for cc in _COS_COEF[-2::-1]:
            cpoly = cpoly * d2 + np.float32(cc)
        w = jnp.where(mask, 0.5 * cpoly + 0.5, 0.0)

        eta = 0.5 * _CUTOFF / _N_RBF
        inv2eta2 = 1.0 / (2.0 * eta * eta)
        inveta2 = 1.0 / (eta * eta)
        centers = np.linspace(_R0, _CUTOFF, _N_RBF)

        a = d2 * np.float32(-inv2eta2)
        rows = []
        cols = []
        for k in range(_N_RBF):
            sk = np.float32(centers[k] * inveta2)
            tk = np.float32(-centers[k] * centers[k] * inv2eta2)
            g = (dist * sk + a) + tk
            wek = w * jnp.exp(g)
            rows.append(jnp.sum(wek, axis=1, keepdims=True))
            cols.append(jnp.sum(wek, axis=0, keepdims=True))
        rowsum = jnp.concatenate(rows, axis=1)
        feat_ref[pl.ds(bi * br, br), :] += rowsum

        @pl.when(bj > bi)
        def _():
            colsum = jnp.concatenate(cols, axis=0).T
            feat_ref[pl.ds(bj * br, br), :] += colsum

    @pl.when(jnp.logical_and(bi == nb - 1, bj == nb - 1))
    def _():
        feats = feat_ref[...]
        h = _silu(jnp.dot(feats, w1t_ref[...],
                          preferred_element_type=jnp.float32) + b1_ref[...])
        h = _silu(jnp.dot(h, w2t_ref[...],
                          preferred_element_type=jnp.float32) + b2_ref[...])
        o = jnp.dot(h, w3t_ref[...], preferred_element_type=jnp.float32)
        total = jnp.sum(o) + np.float32(n) * b3_ref[0, 0]
        out_ref[...] = jnp.full((1, 1), total, jnp.float32)


@jax.jit
def _run(positions, W1, b1, W2, b2, W3, b3):
    n = positions.shape[0]
    br = 256
    nb = n // br
    posT = positions.T
    w1t = W1.T
    w2t = W2.T
    w3t = W3.T
    b1r = b1.reshape(1, -1)
    b2r = b2.reshape(1, -1)
    b3r = b3.reshape(1, 1)

    out = pl.pallas_call(
        functools.partial(_body, br=br, n=n),
        grid=(nb, nb),
        in_specs=[
            pl.BlockSpec((br, 3), lambda i, j: (i, 0)),
            pl.BlockSpec((3, br), lambda i, j: (0, j)),
            pl.BlockSpec(w1t.shape, lambda i, j: (0, 0)),
            pl.BlockSpec(b1r.shape, lambda i, j: (0, 0)),
            pl.BlockSpec(w2t.shape, lambda i, j: (0, 0)),
            pl.BlockSpec(b2r.shape, lambda i, j: (0, 0)),
            pl.BlockSpec(w3t.shape, lambda i, j: (0, 0)),
            pl.BlockSpec(b3r.shape, lambda i, j: (0, 0)),
        ],
        out_specs=pl.BlockSpec((1, 1), lambda i, j: (0, 0)),
        out_shape=jax.ShapeDtypeStruct((1, 1), jnp.float32),
        scratch_shapes=[pltpu.VMEM((n, _N_RBF), jnp.float32)],
        compiler_params=pltpu.CompilerParams(
            dimension_semantics=("arbitrary", "arbitrary"),
        ),
    )(positions, posT, w1t, b1r, w2t, b2r, w3t, b3r)
    return out[0, 0]


def kernel(positions, W1, b1, W2, b2, W3, b3):
    return _run(positions, W1, b1, W2, b2, W3, b3)


# speedup vs baseline: 2.7971x; 1.2581x over previous
import functools

import jax
import jax.numpy as jnp
import numpy as np
from jax.experimental import pallas as pl
from jax.experimental.pallas import tpu as pltpu

_N_RBF = 16
_CUTOFF = 5.0
_R0 = 0.5

_W_COEF = [0.5 + 0.5 * c for c in [1.0]] + [0.5 * c for c in [
    -0.1973920874308756, 0.0064939389052478935,
    -8.545665775343764e-05, 6.024212492804692e-07, -2.6404630605029533e-09,
    7.800022166687994e-12, -1.452987492084503e-14,
]]


def _silu(x):
    return x * jax.nn.sigmoid(x)


def _body(pos_ref, posT_ref, w1t_ref, b1_ref, w2t_ref, b2_ref, w3t_ref,
          b3_ref, out_ref, feat_ref, *, br, n):
    bi = pl.program_id(0)
    bj = pl.program_id(1)
    nb = n // br

    @pl.when(jnp.logical_and(bi == 0, bj == 0))
    def _():
        feat_ref[...] = jnp.zeros((n, _N_RBF), jnp.float32)

    @pl.when(bj >= bi)
    def _():
        pos = pos_ref[...]
        posT = posT_ref[...]

        d2 = jnp.full((br, br), 1e-12, jnp.float32)
        for c in range(3):
            diff = pos[:, c:c + 1] - posT[c:c + 1, :]
            d2 = d2 + diff * diff

        r = jax.lax.rsqrt(d2)
        r = r * (1.5 - (0.5 * d2) * (r * r))
        dist = d2 * r

        mask = (dist > 1e-6) & (dist < _CUTOFF)

        w = jnp.full((br, br), np.float32(_W_COEF[-1]), jnp.float32)
        for cc in _W_COEF[-2::-1]:
            w = w * d2 + np.float32(cc)
        w = jnp.where(mask, w, 0.0)

        eta = 0.5 * _CUTOFF / _N_RBF
        inv2eta2 = 1.0 / (2.0 * eta * eta)
        inveta2 = 1.0 / (eta * eta)
        log2e = float(np.log2(np.e))
        centers = np.linspace(_R0, _CUTOFF, _N_RBF)

        a2 = d2 * np.float32(-inv2eta2 * log2e)
        dl = dist * np.float32(inveta2 * log2e)
        rows = []
        cols = []
        for k in range(_N_RBF):
            ck = np.float32(centers[k])
            t2k = np.float32(-centers[k] * centers[k] * inv2eta2 * log2e)
            g2 = (dl * ck + a2) + t2k
            wek = w * jnp.exp2(g2)
            rows.append(jnp.sum(wek, axis=1, keepdims=True))
            cols.append(jnp.sum(wek, axis=0, keepdims=True))
        rowsum = jnp.concatenate(rows, axis=1)
        feat_ref[pl.ds(bi * br, br), :] += rowsum

        @pl.when(bj > bi)
        def _():
            colsum = jnp.concatenate(cols, axis=0).T
            feat_ref[pl.ds(bj * br, br), :] += colsum

    @pl.when(jnp.logical_and(bi == nb - 1, bj == nb - 1))
    def _():
        feats = feat_ref[...]
        h = _silu(jnp.dot(feats, w1t_ref[...],
                          preferred_element_type=jnp.float32) + b1_ref[...])
        h = _silu(jnp.dot(h, w2t_ref[...],
                          preferred_element_type=jnp.float32) + b2_ref[...])
        o = jnp.dot(h, w3t_ref[...], preferred_element_type=jnp.float32)
        total = jnp.sum(o) + np.float32(n) * b3_ref[0, 0]
        out_ref[...] = jnp.full((1, 1), total, jnp.float32)


@jax.jit
def _run(positions, W1, b1, W2, b2, W3, b3):
    n = positions.shape[0]
    br = 512
    nb = n // br
    posT = positions.T
    w1t = W1.T
    w2t = W2.T
    w3t = W3.T
    b1r = b1.reshape(1, -1)
    b2r = b2.reshape(1, -1)
    b3r = b3.reshape(1, 1)

    out = pl.pallas_call(
        functools.partial(_body, br=br, n=n),
        grid=(nb, nb),
        in_specs=[
            pl.BlockSpec((br, 3), lambda i, j: (i, 0)),
            pl.BlockSpec((3, br), lambda i, j: (0, j)),
            pl.BlockSpec(w1t.shape, lambda i, j: (0, 0)),
            pl.BlockSpec(b1r.shape, lambda i, j: (0, 0)),
            pl.BlockSpec(w2t.shape, lambda i, j: (0, 0)),
            pl.BlockSpec(b2r.shape, lambda i, j: (0, 0)),
            pl.BlockSpec(w3t.shape, lambda i, j: (0, 0)),
            pl.BlockSpec(b3r.shape, lambda i, j: (0, 0)),
        ],
        out_specs=pl.BlockSpec((1, 1), lambda i, j: (0, 0)),
        out_shape=jax.ShapeDtypeStruct((1, 1), jnp.float32),
        scratch_shapes=[pltpu.VMEM((n, _N_RBF), jnp.float32)],
        compiler_params=pltpu.CompilerParams(
            dimension_semantics=("arbitrary", "arbitrary"),
        ),
    )(positions, posT, w1t, b1r, w2t, b2r, w3t, b3r)
    return out[0, 0]


def kernel(positions, W1, b1, W2, b2, W3, b3):
    return _run(positions, W1, b1, W2, b2, W3, b3)


# speedup vs baseline: 3.2925x; 1.1771x over previous
import functools

import jax
import jax.numpy as jnp
import numpy as np
from jax.experimental import pallas as pl
from jax.experimental.pallas import tpu as pltpu

_N_RBF = 16
_CUTOFF = 5.0
_R0 = 0.5

_W_COEF = [0.5 + 0.5 * c for c in [1.0]] + [0.5 * c for c in [
    -0.1973920874308756, 0.0064939389052478935,
    -8.545665775343764e-05, 6.024212492804692e-07, -2.6404630605029533e-09,
    7.800022166687994e-12, -1.452987492084503e-14,
]]


def _silu(x):
    return x * jax.nn.sigmoid(x)


def _body(pos_ref, posT_ref, w1t_ref, b1_ref, w2t_ref, b2_ref, w3t_ref,
          b3_ref, out_ref, feat_ref, *, br, n):
    bi = pl.program_id(0)
    bj = pl.program_id(1)
    nb = n // br

    @pl.when(jnp.logical_and(bi == 0, bj == 0))
    def _():
        feat_ref[...] = jnp.zeros((n, _N_RBF), jnp.float32)

    @pl.when(bj >= bi)
    def _():
        pos = pos_ref[...]
        posT = posT_ref[...]

        gram = jnp.dot(pos, posT * np.float32(-2.0),
                       preferred_element_type=jnp.float32)
        rown = jnp.sum(pos * pos, axis=1, keepdims=True)
        coln = jnp.sum(posT * posT, axis=0, keepdims=True)
        d2 = jnp.maximum((gram + rown) + coln, 1e-12)

        r = jax.lax.rsqrt(d2)
        r = r * (1.5 - (0.5 * d2) * (r * r))
        dist = d2 * r

        mask = (dist > 1e-6) & (dist < _CUTOFF)

        w = jnp.full((br, br), np.float32(_W_COEF[-1]), jnp.float32)
        for cc in _W_COEF[-2::-1]:
            w = w * d2 + np.float32(cc)
        w = jnp.where(mask, jnp.maximum(w, 0.0), 0.0)

        eta = 0.5 * _CUTOFF / _N_RBF
        inv2eta2 = 1.0 / (2.0 * eta * eta)
        inveta2 = 1.0 / (eta * eta)
        log2e = float(np.log2(np.e))
        centers = np.linspace(_R0, _CUTOFF, _N_RBF)
        dc = float(centers[1] - centers[0])

        lw = jnp.log2(w)
        g2 = lw + (dist - np.float32(centers[0])) ** 2 * \
            np.float32(-inv2eta2 * log2e)
        step = dist * np.float32(dc * inveta2 * log2e) + \
            np.float32(-(centers[1] ** 2 - centers[0] ** 2) * inv2eta2 * log2e)
        step2 = np.float32(-2.0 * dc * dc * inv2eta2 * log2e)
        rows = []
        cols = []
        for k in range(_N_RBF):
            wek = jnp.exp2(g2)
            rows.append(jnp.sum(wek, axis=1, keepdims=True))
            cols.append(jnp.sum(wek, axis=0, keepdims=True))
            if k + 1 < _N_RBF:
                g2 = g2 + step
                step = step + step2
        rowsum = jnp.concatenate(rows, axis=1)
        feat_ref[pl.ds(bi * br, br), :] += rowsum

        @pl.when(bj > bi)
        def _():
            colsum = jnp.concatenate(cols, axis=0).T
            feat_ref[pl.ds(bj * br, br), :] += colsum

    @pl.when(jnp.logical_and(bi == nb - 1, bj == nb - 1))
    def _():
        feats = feat_ref[...]
        h = _silu(jnp.dot(feats, w1t_ref[...],
                          preferred_element_type=jnp.float32) + b1_ref[...])
        h = _silu(jnp.dot(h, w2t_ref[...],
                          preferred_element_type=jnp.float32) + b2_ref[...])
        o = jnp.dot(h, w3t_ref[...], preferred_element_type=jnp.float32)
        total = jnp.sum(o) + np.float32(n) * b3_ref[0, 0]
        out_ref[...] = jnp.full((1, 1), total, jnp.float32)


@jax.jit
def _run(positions, W1, b1, W2, b2, W3, b3):
    n = positions.shape[0]
    br = 512
    nb = n // br
    posT = positions.T
    w1t = W1.T
    w2t = W2.T
    w3t = W3.T
    b1r = b1.reshape(1, -1)
    b2r = b2.reshape(1, -1)
    b3r = b3.reshape(1, 1)

    out = pl.pallas_call(
        functools.partial(_body, br=br, n=n),
        grid=(nb, nb),
        in_specs=[
            pl.BlockSpec((br, 3), lambda i, j: (i, 0)),
            pl.BlockSpec((3, br), lambda i, j: (0, j)),
            pl.BlockSpec(w1t.shape, lambda i, j: (0, 0)),
            pl.BlockSpec(b1r.shape, lambda i, j: (0, 0)),
            pl.BlockSpec(w2t.shape, lambda i, j: (0, 0)),
            pl.BlockSpec(b2r.shape, lambda i, j: (0, 0)),
            pl.BlockSpec(w3t.shape, lambda i, j: (0, 0)),
            pl.BlockSpec(b3r.shape, lambda i, j: (0, 0)),
        ],
        out_specs=pl.BlockSpec((1, 1), lambda i, j: (0, 0)),
        out_shape=jax.ShapeDtypeStruct((1, 1), jnp.float32),
        scratch_shapes=[pltpu.VMEM((n, _N_RBF), jnp.float32)],
        compiler_params=pltpu.CompilerParams(
            dimension_semantics=("arbitrary", "arbitrary"),
        ),
    )(positions, posT, w1t, b1r, w2t, b2r, w3t, b3r)
    return out[0, 0]


def kernel(positions, W1, b1, W2, b2, W3, b3):
    return _run(positions, W1, b1, W2, b2, W3, b3)
